# baseline (speedup 1.0000x reference)
import jax
import jax.numpy as jnp
from jax import lax
from jax.experimental import pallas as pl
from jax.experimental.pallas import tpu as pltpu, tpu_sc as plsc

BATCH = 16384
DIM = 64
NSC = 2
NSUB = 16
J_PER_SC = DIM // NSC
B_PER_SUB = BATCH // NSUB
NENT = 1000000
RELW = 1024
ROWBUF = NENT + RELW
IDX_CHUNK = 128
NCH = B_PER_SUB // IDX_CHUNK
GW = 3 * IDX_CHUNK


def _sc_body(ent_hbm, rel_hbm, tail_hbm, h_hbm, r_hbm, t_hbm, p0_hbm, p1_hbm,
             ent0, ent1, rel0, rel1, hidx, tidx, ridx, gbuf, tailv0, tailv1, acc,
             sem_r0, sem_r1, sem_g0, sem_g1):
    c = lax.axis_index("c")
    s = lax.axis_index("s")
    jbase = c * J_PER_SC
    b0 = s * B_PER_SUB

    ent_bufs = (ent0, ent1)
    rel_bufs = (rel0, rel1)
    tail_bufs = (tailv0, tailv1)
    row_sems = (sem_r0, sem_r1)
    g_sems = (sem_g0, sem_g1)

    row_slices = [(i * 124928, 124928) for i in range(8)]
    row_slices += [(999424, 512)]
    TAIL_OFF = 999936
    TAILW = 64

    def issue_row(jj, buf, rbuf, slot, sem):
        for i, (off, ln) in enumerate(row_slices):
            @pl.when(s == i)
            def _(off=off, ln=ln):
                pltpu.async_copy(ent_hbm.at[jbase + jj, pl.ds(off, ln)],
                                 buf.at[pl.ds(off, ln)], sem)

        @pl.when(s == 9)
        def _():
            pltpu.async_copy(rel_hbm.at[jbase + jj], rbuf, sem)

        @pl.when(s == 11)
        def _():
            pltpu.async_copy(tail_hbm.at[jbase + jj], tail_bufs[slot], sem)

    def wait_row(jj, buf, rbuf, slot, sem):
        for i, (off, ln) in enumerate(row_slices):
            @pl.when(s == i)
            def _(off=off, ln=ln):
                pltpu.make_async_copy(ent_hbm.at[0, pl.ds(0, ln)],
                                      buf.at[pl.ds(0, ln)], sem).wait()

        @pl.when(s == 9)
        def _():
            pltpu.make_async_copy(rel_hbm.at[0], rbuf, sem).wait()

        @pl.when(s == 11)
        def _():
            pltpu.make_async_copy(tail_hbm.at[0], tail_bufs[slot], sem).wait()
            pltpu.sync_copy(tail_bufs[slot], buf.at[pl.ds(TAIL_OFF, TAILW)])

    pltpu.sync_copy(h_hbm.at[pl.ds(b0, B_PER_SUB)], hidx)
    pltpu.sync_copy(t_hbm.at[pl.ds(b0, B_PER_SUB)], tidx)
    pltpu.sync_copy(r_hbm.at[pl.ds(b0, B_PER_SUB)], ridx)
    for i in range(B_PER_SUB // 16):
        d = pl.ds(i * 16, 16)
        acc[d] = jnp.zeros((16,), jnp.float32)
    issue_row(0, ent0, rel0, 0, sem_r0)
    issue_row(1, ent1, rel1, 1, sem_r1)

    def issue_chunk(buf, rbuf, ch, slot):
        d = pl.ds(ch * IDX_CHUNK, IDX_CHUNK)
        base = slot * GW
        sem = g_sems[slot]
        pltpu.async_copy(buf.at[hidx.at[d]], gbuf.at[pl.ds(base, IDX_CHUNK)], sem)
        pltpu.async_copy(buf.at[tidx.at[d]],
                         gbuf.at[pl.ds(base + IDX_CHUNK, IDX_CHUNK)], sem)
        pltpu.async_copy(rbuf.at[ridx.at[d]],
                         gbuf.at[pl.ds(base + 2 * IDX_CHUNK, IDX_CHUNK)], sem)

    def wait_chunk(slot):
        pltpu.make_async_copy(ent_hbm.at[0, pl.ds(0, GW)],
                              gbuf.at[pl.ds(slot * GW, GW)], g_sems[slot]).wait()

    def process_j(jj, buf_id):
        buf = ent_bufs[buf_id]
        rbuf = rel_bufs[buf_id]
        wait_row(jj, buf, rbuf, buf_id, row_sems[buf_id])
        plsc.subcore_barrier()

        issue_chunk(buf, rbuf, 0, 0)
        for ch in range(NCH):
            slot = ch % 2
            if ch + 1 < NCH:
                issue_chunk(buf, rbuf, ch + 1, (ch + 1) % 2)
            wait_chunk(slot)
            base = slot * GW
            for i in range(IDX_CHUNK // 16):
                hv = gbuf[pl.ds(base + i * 16, 16)]
                tv = gbuf[pl.ds(base + IDX_CHUNK + i * 16, 16)]
                rv = gbuf[pl.ds(base + 2 * IDX_CHUNK + i * 16, 16)]
                dd = hv + rv - tv
                a = pl.ds(ch * IDX_CHUNK + i * 16, 16)
                acc[a] = acc[a] + dd * dd

        plsc.subcore_barrier()

        @pl.when(jj + 2 < J_PER_SC)
        def _():
            issue_row(jj + 2, buf, rbuf, buf_id, row_sems[buf_id])

    def loop2(i, carry):
        process_j(i * 2, 0)
        process_j(i * 2 + 1, 1)
        return carry

    lax.fori_loop(0, J_PER_SC // 2, loop2, 0)

    @pl.when(c == 0)
    def _():
        pltpu.sync_copy(acc, p0_hbm.at[pl.ds(b0, B_PER_SUB)])

    @pl.when(c == 1)
    def _():
        pltpu.sync_copy(acc, p1_hbm.at[pl.ds(b0, B_PER_SUB)])


def _combine_body(p0_ref, p1_ref, o_ref):
    o_ref[...] = jnp.sqrt(p0_ref[...] + p1_ref[...] + 1e-12)


@jax.jit
def _transe(emb_ent, emb_rel, h, r, t):
    ent_t = emb_ent.T
    rel_t = jnp.pad(emb_rel.T, ((0, 0), (0, RELW - emb_rel.shape[0])))
    tail_t = emb_ent[NENT - 64:, :].T
    mesh = plsc.VectorSubcoreMesh(core_axis_name="c", subcore_axis_name="s",
                                  num_cores=NSC, num_subcores=NSUB)
    f = pl.kernel(
        _sc_body,
        out_type=(jax.ShapeDtypeStruct((BATCH,), jnp.float32),
                  jax.ShapeDtypeStruct((BATCH,), jnp.float32)),
        mesh=mesh,
        compiler_params=pltpu.CompilerParams(
            use_tc_tiling_on_sc=True,
            needs_layout_passes=False,
        ),
        scratch_types=[
            pltpu.VMEM_SHARED((NENT,), jnp.float32),
            pltpu.VMEM_SHARED((NENT,), jnp.float32),
            pltpu.VMEM_SHARED((RELW,), jnp.float32),
            pltpu.VMEM_SHARED((RELW,), jnp.float32),
            pltpu.VMEM((B_PER_SUB,), jnp.int32),
            pltpu.VMEM((B_PER_SUB,), jnp.int32),
            pltpu.VMEM((B_PER_SUB,), jnp.int32),
            pltpu.VMEM((2 * GW,), jnp.float32),
            pltpu.VMEM((64,), jnp.float32),
            pltpu.VMEM((64,), jnp.float32),
            pltpu.VMEM((B_PER_SUB,), jnp.float32),
            pltpu.SemaphoreType.DMA,
            pltpu.SemaphoreType.DMA,
            pltpu.SemaphoreType.DMA,
            pltpu.SemaphoreType.DMA,
        ],
    )
    p0, p1 = f(ent_t, rel_t, tail_t, h, r, t)
    out = pl.pallas_call(
        _combine_body,
        out_shape=jax.ShapeDtypeStruct((128, 128), jnp.float32),
    )(p0.reshape(128, 128), p1.reshape(128, 128))
    return out.reshape(BATCH)


def kernel(emb_ent, emb_rel, h, r, t):
    return _transe(emb_ent, emb_rel,
                   h.astype(jnp.int32), r.astype(jnp.int32),
                   t.astype(jnp.int32))


# speedup vs baseline: 1.0093x; 1.0093x over previous
import jax
import jax.numpy as jnp
from jax import lax
from jax.experimental import pallas as pl
from jax.experimental.pallas import tpu as pltpu, tpu_sc as plsc

BATCH = 16384
DIM = 64
NSC = 2
NSUB = 16
J_PER_SC = DIM // NSC
B_PER_SUB = BATCH // NSUB
NENT = 1000000
RELW = 1024
ROWBUF = NENT + RELW
IDX_CHUNK = 128
NCH = B_PER_SUB // IDX_CHUNK
GW = 3 * IDX_CHUNK


def _sc_body(ent_hbm, rel_hbm, tail_hbm, h_hbm, r_hbm, t_hbm, p0_hbm, p1_hbm,
             ent0, ent1, rel0, rel1, hidx, tidx, ridx, gbuf, tailv0, tailv1, acc,
             sem_r0, sem_r1, sem_g0, sem_g1):
    c = lax.axis_index("c")
    s = lax.axis_index("s")
    jbase = c * J_PER_SC
    b0 = s * B_PER_SUB

    ent_bufs = (ent0, ent1)
    rel_bufs = (rel0, rel1)
    tail_bufs = (tailv0, tailv1)
    row_sems = (sem_r0, sem_r1)
    g_sems = (sem_g0, sem_g1)

    row_slices = [(i * 124928, 124928) for i in range(8)]
    row_slices += [(999424, 512)]
    TAIL_OFF = 999936
    TAILW = 64

    def issue_row(jj, buf, rbuf, slot, sem):
        for i, (off, ln) in enumerate(row_slices):
            @pl.when(s == i)
            def _(off=off, ln=ln):
                pltpu.async_copy(ent_hbm.at[jbase + jj, pl.ds(off, ln)],
                                 buf.at[pl.ds(off, ln)], sem)

        @pl.when(s == 9)
        def _():
            pltpu.async_copy(rel_hbm.at[jbase + jj], rbuf, sem)

        @pl.when(s == 11)
        def _():
            pltpu.async_copy(tail_hbm.at[jbase + jj], tail_bufs[slot], sem)

    def wait_row(jj, buf, rbuf, slot, sem):
        for i, (off, ln) in enumerate(row_slices):
            @pl.when(s == i)
            def _(off=off, ln=ln):
                pltpu.make_async_copy(ent_hbm.at[0, pl.ds(0, ln)],
                                      buf.at[pl.ds(0, ln)], sem).wait()

        @pl.when(s == 9)
        def _():
            pltpu.make_async_copy(rel_hbm.at[0], rbuf, sem).wait()

        @pl.when(s == 11)
        def _():
            pltpu.make_async_copy(tail_hbm.at[0], tail_bufs[slot], sem).wait()
            pltpu.sync_copy(tail_bufs[slot], buf.at[pl.ds(TAIL_OFF, TAILW)])

    issue_row(0, ent0, rel0, 0, sem_r0)
    issue_row(1, ent1, rel1, 1, sem_r1)
    pltpu.async_copy(h_hbm.at[pl.ds(b0, B_PER_SUB)], hidx, sem_g0)
    pltpu.async_copy(t_hbm.at[pl.ds(b0, B_PER_SUB)], tidx, sem_g0)
    pltpu.async_copy(r_hbm.at[pl.ds(b0, B_PER_SUB)], ridx, sem_g0)
    for i in range(B_PER_SUB // 16):
        d = pl.ds(i * 16, 16)
        acc[d] = jnp.zeros((16,), jnp.float32)
    pltpu.make_async_copy(h_hbm.at[pl.ds(0, B_PER_SUB)], hidx, sem_g0).wait()
    pltpu.make_async_copy(h_hbm.at[pl.ds(0, B_PER_SUB)], tidx, sem_g0).wait()
    pltpu.make_async_copy(h_hbm.at[pl.ds(0, B_PER_SUB)], ridx, sem_g0).wait()

    def issue_chunk(buf, rbuf, ch, slot):
        d = pl.ds(ch * IDX_CHUNK, IDX_CHUNK)
        base = slot * GW
        sem = g_sems[slot]
        pltpu.async_copy(buf.at[hidx.at[d]], gbuf.at[pl.ds(base, IDX_CHUNK)], sem)
        pltpu.async_copy(buf.at[tidx.at[d]],
                         gbuf.at[pl.ds(base + IDX_CHUNK, IDX_CHUNK)], sem)
        pltpu.async_copy(rbuf.at[ridx.at[d]],
                         gbuf.at[pl.ds(base + 2 * IDX_CHUNK, IDX_CHUNK)], sem)

    def wait_chunk(slot):
        pltpu.make_async_copy(ent_hbm.at[0, pl.ds(0, GW)],
                              gbuf.at[pl.ds(slot * GW, GW)], g_sems[slot]).wait()

    def process_j(jj, buf_id):
        buf = ent_bufs[buf_id]
        rbuf = rel_bufs[buf_id]
        wait_row(jj, buf, rbuf, buf_id, row_sems[buf_id])
        plsc.subcore_barrier()

        issue_chunk(buf, rbuf, 0, 0)
        for ch in range(NCH):
            slot = ch % 2
            if ch + 1 < NCH:
                issue_chunk(buf, rbuf, ch + 1, (ch + 1) % 2)
            wait_chunk(slot)
            base = slot * GW
            for i in range(IDX_CHUNK // 16):
                hv = gbuf[pl.ds(base + i * 16, 16)]
                tv = gbuf[pl.ds(base + IDX_CHUNK + i * 16, 16)]
                rv = gbuf[pl.ds(base + 2 * IDX_CHUNK + i * 16, 16)]
                dd = hv + rv - tv
                a = pl.ds(ch * IDX_CHUNK + i * 16, 16)
                acc[a] = acc[a] + dd * dd

        plsc.subcore_barrier()

        @pl.when(jj + 2 < J_PER_SC)
        def _():
            issue_row(jj + 2, buf, rbuf, buf_id, row_sems[buf_id])

    def loop2(i, carry):
        process_j(i * 2, 0)
        process_j(i * 2 + 1, 1)
        return carry

    lax.fori_loop(0, J_PER_SC // 2, loop2, 0)

    @pl.when(c == 0)
    def _():
        pltpu.sync_copy(acc, p0_hbm.at[pl.ds(b0, B_PER_SUB)])

    @pl.when(c == 1)
    def _():
        pltpu.sync_copy(acc, p1_hbm.at[pl.ds(b0, B_PER_SUB)])


def _combine_body(p0_ref, p1_ref, o_ref):
    o_ref[...] = jnp.sqrt(p0_ref[...] + p1_ref[...] + 1e-12)


@jax.jit
def _transe(emb_ent, emb_rel, h, r, t):
    ent_t = emb_ent.T
    rel_t = jnp.pad(emb_rel.T, ((0, 0), (0, RELW - emb_rel.shape[0])))
    tail_t = emb_ent[NENT - 64:, :].T
    mesh = plsc.VectorSubcoreMesh(core_axis_name="c", subcore_axis_name="s",
                                  num_cores=NSC, num_subcores=NSUB)
    f = pl.kernel(
        _sc_body,
        out_type=(jax.ShapeDtypeStruct((BATCH,), jnp.float32),
                  jax.ShapeDtypeStruct((BATCH,), jnp.float32)),
        mesh=mesh,
        compiler_params=pltpu.CompilerParams(
            use_tc_tiling_on_sc=True,
            needs_layout_passes=False,
        ),
        scratch_types=[
            pltpu.VMEM_SHARED((NENT,), jnp.float32),
            pltpu.VMEM_SHARED((NENT,), jnp.float32),
            pltpu.VMEM_SHARED((RELW,), jnp.float32),
            pltpu.VMEM_SHARED((RELW,), jnp.float32),
            pltpu.VMEM((B_PER_SUB,), jnp.int32),
            pltpu.VMEM((B_PER_SUB,), jnp.int32),
            pltpu.VMEM((B_PER_SUB,), jnp.int32),
            pltpu.VMEM((2 * GW,), jnp.float32),
            pltpu.VMEM((64,), jnp.float32),
            pltpu.VMEM((64,), jnp.float32),
            pltpu.VMEM((B_PER_SUB,), jnp.float32),
            pltpu.SemaphoreType.DMA,
            pltpu.SemaphoreType.DMA,
            pltpu.SemaphoreType.DMA,
            pltpu.SemaphoreType.DMA,
        ],
    )
    p0, p1 = f(ent_t, rel_t, tail_t, h, r, t)
    out = pl.pallas_call(
        _combine_body,
        out_shape=jax.ShapeDtypeStruct((128, 128), jnp.float32),
    )(p0.reshape(128, 128), p1.reshape(128, 128))
    return out.reshape(BATCH)


def kernel(emb_ent, emb_rel, h, r, t):
    return _transe(emb_ent, emb_rel,
                   h.astype(jnp.int32), r.astype(jnp.int32),
                   t.astype(jnp.int32))


# speedup vs baseline: 1.0101x; 1.0008x over previous
import jax
import jax.numpy as jnp
from jax import lax
from jax.experimental import pallas as pl
from jax.experimental.pallas import tpu as pltpu, tpu_sc as plsc

BATCH = 16384
DIM = 64
NSC = 2
NSUB = 16
J_PER_SC = DIM // NSC
B_PER_SUB = BATCH // NSUB
NENT = 1000000
RELW = 1024
IDX_CHUNK = 128
NCH = B_PER_SUB // IDX_CHUNK
GW = 3 * IDX_CHUNK


def _sc_body(ent_hbm, rel_hbm, tail_hbm, h_hbm, r_hbm, t_hbm, p0_hbm, p1_hbm,
             ent0, ent1, rel0, rel1, hidx, tidx, ridx, gbuf, tailv0, tailv1, acc,
             sem_r0, sem_r1, sem_g0, sem_g1):
    c = lax.axis_index("c")
    s = lax.axis_index("s")
    jbase = c * J_PER_SC
    b0 = s * B_PER_SUB

    ent_bufs = (ent0, ent1)
    rel_bufs = (rel0, rel1)
    tail_bufs = (tailv0, tailv1)
    row_sems = (sem_r0, sem_r1)
    g_sems = (sem_g0, sem_g1)

    row_slices = [(i * 124928, 124928) for i in range(8)]
    row_slices += [(999424, 512)]
    TAIL_OFF = 999936
    TAILW = 64

    def issue_row(jj, buf, rbuf, slot, sem):
        for i, (off, ln) in enumerate(row_slices):
            @pl.when(s == i)
            def _(off=off, ln=ln):
                pltpu.async_copy(ent_hbm.at[jbase + jj, pl.ds(off, ln)],
                                 buf.at[pl.ds(off, ln)], sem)

        @pl.when(s == 9)
        def _():
            pltpu.async_copy(rel_hbm.at[jbase + jj], rbuf, sem)

        @pl.when(s == 11)
        def _():
            pltpu.async_copy(tail_hbm.at[jbase + jj], tail_bufs[slot], sem)

    def wait_row(jj, buf, rbuf, slot, sem):
        for i, (off, ln) in enumerate(row_slices):
            @pl.when(s == i)
            def _(off=off, ln=ln):
                pltpu.make_async_copy(ent_hbm.at[0, pl.ds(0, ln)],
                                      buf.at[pl.ds(0, ln)], sem).wait()

        @pl.when(s == 9)
        def _():
            pltpu.make_async_copy(rel_hbm.at[0], rbuf, sem).wait()

        @pl.when(s == 11)
        def _():
            pltpu.make_async_copy(tail_hbm.at[0], tail_bufs[slot], sem).wait()
            pltpu.sync_copy(tail_bufs[slot], buf.at[pl.ds(TAIL_OFF, TAILW)])

    issue_row(0, ent0, rel0, 0, sem_r0)
    issue_row(1, ent1, rel1, 1, sem_r1)
    pltpu.async_copy(h_hbm.at[pl.ds(b0, B_PER_SUB)], hidx, sem_g0)
    pltpu.async_copy(t_hbm.at[pl.ds(b0, B_PER_SUB)], tidx, sem_g0)
    pltpu.async_copy(r_hbm.at[pl.ds(b0, B_PER_SUB)], ridx, sem_g0)
    for i in range(B_PER_SUB // 16):
        d = pl.ds(i * 16, 16)
        acc[d] = jnp.zeros((16,), jnp.float32)
    pltpu.make_async_copy(h_hbm.at[pl.ds(0, B_PER_SUB)], hidx, sem_g0).wait()
    pltpu.make_async_copy(h_hbm.at[pl.ds(0, B_PER_SUB)], tidx, sem_g0).wait()
    pltpu.make_async_copy(h_hbm.at[pl.ds(0, B_PER_SUB)], ridx, sem_g0).wait()

    def issue_chunk(buf, rbuf, ch, slot):
        d = pl.ds(ch * IDX_CHUNK, IDX_CHUNK)
        base = slot * GW
        sem = g_sems[slot]
        pltpu.async_copy(buf.at[hidx.at[d]], gbuf.at[pl.ds(base, IDX_CHUNK)], sem)
        pltpu.async_copy(buf.at[tidx.at[d]],
                         gbuf.at[pl.ds(base + IDX_CHUNK, IDX_CHUNK)], sem)
        pltpu.async_copy(rbuf.at[ridx.at[d]],
                         gbuf.at[pl.ds(base + 2 * IDX_CHUNK, IDX_CHUNK)], sem)

    def wait_chunk(slot):
        pltpu.make_async_copy(ent_hbm.at[0, pl.ds(0, GW)],
                              gbuf.at[pl.ds(slot * GW, GW)], g_sems[slot]).wait()

    def process_j(jj, buf_id):
        buf = ent_bufs[buf_id]
        rbuf = rel_bufs[buf_id]
        wait_row(jj, buf, rbuf, buf_id, row_sems[buf_id])
        plsc.subcore_barrier()

        issue_chunk(buf, rbuf, 0, 0)
        for ch in range(NCH):
            slot = ch % 2
            if ch + 1 < NCH:
                issue_chunk(buf, rbuf, ch + 1, (ch + 1) % 2)
            wait_chunk(slot)
            base = slot * GW
            for i in range(IDX_CHUNK // 16):
                hv = gbuf[pl.ds(base + i * 16, 16)]
                tv = gbuf[pl.ds(base + IDX_CHUNK + i * 16, 16)]
                rv = gbuf[pl.ds(base + 2 * IDX_CHUNK + i * 16, 16)]
                dd = hv + rv - tv
                a = pl.ds(ch * IDX_CHUNK + i * 16, 16)
                acc[a] = acc[a] + dd * dd

        plsc.subcore_barrier()

        @pl.when(jj + 2 < J_PER_SC)
        def _():
            issue_row(jj + 2, buf, rbuf, buf_id, row_sems[buf_id])

    def loop2(i, carry):
        process_j(i * 2, 0)
        process_j(i * 2 + 1, 1)
        return carry

    lax.fori_loop(0, J_PER_SC // 2, loop2, 0)

    @pl.when(c == 0)
    def _():
        pltpu.sync_copy(acc, p0_hbm.at[pl.ds(b0, B_PER_SUB)])

    @pl.when(c == 1)
    def _():
        pltpu.sync_copy(acc, p1_hbm.at[pl.ds(b0, B_PER_SUB)])


def _combine_body(p0_ref, p1_ref, o_ref):
    o_ref[...] = jnp.sqrt(p0_ref[...] + p1_ref[...] + 1e-12)


@jax.jit
def _transe(emb_ent, emb_rel, h, r, t):
    ent_t = emb_ent.T
    rel_t = jnp.pad(emb_rel.T, ((0, 0), (0, RELW - emb_rel.shape[0])))
    tail_t = emb_ent[NENT - 64:, :].T
    mesh = plsc.VectorSubcoreMesh(core_axis_name="c", subcore_axis_name="s",
                                  num_cores=NSC, num_subcores=NSUB)
    f = pl.kernel(
        _sc_body,
        out_type=(jax.ShapeDtypeStruct((BATCH,), jnp.float32),
                  jax.ShapeDtypeStruct((BATCH,), jnp.float32)),
        mesh=mesh,
        compiler_params=pltpu.CompilerParams(
            use_tc_tiling_on_sc=True,
            needs_layout_passes=False,
        ),
        scratch_types=[
            pltpu.VMEM_SHARED((NENT,), jnp.float32),
            pltpu.VMEM_SHARED((NENT,), jnp.float32),
            pltpu.VMEM_SHARED((RELW,), jnp.float32),
            pltpu.VMEM_SHARED((RELW,), jnp.float32),
            pltpu.VMEM((B_PER_SUB,), jnp.int32),
            pltpu.VMEM((B_PER_SUB,), jnp.int32),
            pltpu.VMEM((B_PER_SUB,), jnp.int32),
            pltpu.VMEM((2 * GW,), jnp.float32),
            pltpu.VMEM((64,), jnp.float32),
            pltpu.VMEM((64,), jnp.float32),
            pltpu.VMEM((B_PER_SUB,), jnp.float32),
            pltpu.SemaphoreType.DMA,
            pltpu.SemaphoreType.DMA,
            pltpu.SemaphoreType.DMA,
            pltpu.SemaphoreType.DMA,
        ],
    )
    p0, p1 = f(ent_t, rel_t, tail_t, h, r, t)
    out = pl.pallas_call(
        _combine_body,
        out_shape=jax.ShapeDtypeStruct((128, 128), jnp.float32),
    )(p0.reshape(128, 128), p1.reshape(128, 128))
    return out.reshape(BATCH)


def kernel(emb_ent, emb_rel, h, r, t):
    return _transe(emb_ent, emb_rel,
                   h.astype(jnp.int32), r.astype(jnp.int32),
                   t.astype(jnp.int32))


# speedup vs baseline: 1.0113x; 1.0012x over previous
import jax
import jax.numpy as jnp
from jax import lax
from jax.experimental import pallas as pl
from jax.experimental.pallas import tpu as pltpu, tpu_sc as plsc

BATCH = 16384
DIM = 64
NSC = 2
NSUB = 16
J_PER_SC = DIM // NSC
B_PER_SUB = BATCH // NSUB
NENT = 1000000
RELW = 1024
IDX_CHUNK = 128
NCH = B_PER_SUB // IDX_CHUNK
GW = 3 * IDX_CHUNK


def _sc_body(ent_hbm, rel_hbm, tail_hbm, h_hbm, r_hbm, t_hbm, p0_hbm, p1_hbm,
             ent0, ent1, rel0, rel1, hidx, tidx, ridx, gbuf, tailv0, tailv1, acc,
             sem_r0, sem_r1, sem_g0, sem_g1):
    c = lax.axis_index("c")
    s = lax.axis_index("s")
    jbase = c * J_PER_SC
    b0 = s * B_PER_SUB

    ent_bufs = (ent0, ent1)
    rel_bufs = (rel0, rel1)
    tail_bufs = (tailv0, tailv1)
    row_sems = (sem_r0, sem_r1)
    g_sems = (sem_g0, sem_g1)

    row_slices = [(i * 124928, 124928) for i in range(8)]
    row_slices += [(999424, 512)]
    TAIL_OFF = 999936
    TAILW = 64

    def issue_row(jj, buf, rbuf, slot, sem):
        for i, (off, ln) in enumerate(row_slices):
            @pl.when(s == i)
            def _(off=off, ln=ln):
                pltpu.async_copy(ent_hbm.at[jbase + jj, pl.ds(off, ln)],
                                 buf.at[pl.ds(off, ln)], sem)

        @pl.when(s == 9)
        def _():
            pltpu.async_copy(rel_hbm.at[jbase + jj], rbuf, sem)

        @pl.when(s == 11)
        def _():
            pltpu.async_copy(tail_hbm.at[jbase + jj], tail_bufs[slot], sem)

    def wait_row(jj, buf, rbuf, slot, sem):
        for i, (off, ln) in enumerate(row_slices):
            @pl.when(s == i)
            def _(off=off, ln=ln):
                pltpu.make_async_copy(ent_hbm.at[0, pl.ds(0, ln)],
                                      buf.at[pl.ds(0, ln)], sem).wait()

        @pl.when(s == 9)
        def _():
            pltpu.make_async_copy(rel_hbm.at[0], rbuf, sem).wait()

        @pl.when(s == 11)
        def _():
            pltpu.make_async_copy(tail_hbm.at[0], tail_bufs[slot], sem).wait()
            pltpu.sync_copy(tail_bufs[slot], buf.at[pl.ds(TAIL_OFF, TAILW)])

    issue_row(0, ent0, rel0, 0, sem_r0)
    issue_row(1, ent1, rel1, 1, sem_r1)
    pltpu.async_copy(h_hbm.at[pl.ds(b0, B_PER_SUB)], hidx, sem_g0)
    pltpu.async_copy(t_hbm.at[pl.ds(b0, B_PER_SUB)], tidx, sem_g0)
    pltpu.async_copy(r_hbm.at[pl.ds(b0, B_PER_SUB)], ridx, sem_g0)
    for i in range(B_PER_SUB // 16):
        d = pl.ds(i * 16, 16)
        acc[d] = jnp.zeros((16,), jnp.float32)
    pltpu.make_async_copy(h_hbm.at[pl.ds(0, B_PER_SUB)], hidx, sem_g0).wait()
    pltpu.make_async_copy(h_hbm.at[pl.ds(0, B_PER_SUB)], tidx, sem_g0).wait()
    pltpu.make_async_copy(h_hbm.at[pl.ds(0, B_PER_SUB)], ridx, sem_g0).wait()

    def issue_chunk(buf, rbuf, ch, slot):
        d = pl.ds(ch * IDX_CHUNK, IDX_CHUNK)
        base = slot * GW
        sem = g_sems[slot]
        pltpu.async_copy(buf.at[hidx.at[d]], gbuf.at[pl.ds(base, IDX_CHUNK)], sem)
        pltpu.async_copy(buf.at[tidx.at[d]],
                         gbuf.at[pl.ds(base + IDX_CHUNK, IDX_CHUNK)], sem)
        pltpu.async_copy(rbuf.at[ridx.at[d]],
                         gbuf.at[pl.ds(base + 2 * IDX_CHUNK, IDX_CHUNK)], sem)

    def wait_chunk(slot):
        pltpu.make_async_copy(ent_hbm.at[0, pl.ds(0, GW)],
                              gbuf.at[pl.ds(slot * GW, GW)], g_sems[slot]).wait()

    def process_j(jj, buf_id):
        buf = ent_bufs[buf_id]
        rbuf = rel_bufs[buf_id]
        wait_row(jj, buf, rbuf, buf_id, row_sems[buf_id])
        plsc.subcore_barrier()

        issue_chunk(buf, rbuf, 0, 0)
        for ch in range(NCH):
            slot = ch % 2
            if ch + 1 < NCH:
                issue_chunk(buf, rbuf, ch + 1, (ch + 1) % 2)
            wait_chunk(slot)
            base = slot * GW
            for i in range(IDX_CHUNK // 16):
                hv = gbuf[pl.ds(base + i * 16, 16)]
                tv = gbuf[pl.ds(base + IDX_CHUNK + i * 16, 16)]
                rv = gbuf[pl.ds(base + 2 * IDX_CHUNK + i * 16, 16)]
                dd = hv + rv - tv
                a = pl.ds(ch * IDX_CHUNK + i * 16, 16)
                acc[a] = acc[a] + dd * dd

        plsc.subcore_barrier()

        @pl.when(jj + 2 < J_PER_SC)
        def _():
            issue_row(jj + 2, buf, rbuf, buf_id, row_sems[buf_id])

    def loop2(i, carry):
        process_j(i * 2, 0)
        process_j(i * 2 + 1, 1)
        return carry

    lax.fori_loop(0, J_PER_SC // 2, loop2, 0)

    @pl.when(c == 0)
    def _():
        pltpu.sync_copy(acc, p0_hbm.at[pl.ds(b0, B_PER_SUB)])

    @pl.when(c == 1)
    def _():
        pltpu.sync_copy(acc, p1_hbm.at[pl.ds(b0, B_PER_SUB)])


def _combine_body(p0_ref, p1_ref, o_ref):
    o_ref[...] = jnp.sqrt(p0_ref[...] + p1_ref[...] + 1e-12)


@jax.jit
def _transe(emb_ent, emb_rel, h, r, t):
    ent_t = emb_ent.T
    rel_t = jnp.pad(emb_rel.T, ((0, 0), (0, RELW - emb_rel.shape[0])))
    tail_t = emb_ent[NENT - 64:, :].T
    mesh = plsc.VectorSubcoreMesh(core_axis_name="c", subcore_axis_name="s",
                                  num_cores=NSC, num_subcores=NSUB)
    f = pl.kernel(
        _sc_body,
        out_type=(jax.ShapeDtypeStruct((BATCH,), jnp.float32),
                  jax.ShapeDtypeStruct((BATCH,), jnp.float32)),
        mesh=mesh,
        compiler_params=pltpu.CompilerParams(
            use_tc_tiling_on_sc=True,
            needs_layout_passes=False,
            skip_device_barrier=True,
        ),
        scratch_types=[
            pltpu.VMEM_SHARED((NENT,), jnp.float32),
            pltpu.VMEM_SHARED((NENT,), jnp.float32),
            pltpu.VMEM_SHARED((RELW,), jnp.float32),
            pltpu.VMEM_SHARED((RELW,), jnp.float32),
            pltpu.VMEM((B_PER_SUB,), jnp.int32),
            pltpu.VMEM((B_PER_SUB,), jnp.int32),
            pltpu.VMEM((B_PER_SUB,), jnp.int32),
            pltpu.VMEM((2 * GW,), jnp.float32),
            pltpu.VMEM((64,), jnp.float32),
            pltpu.VMEM((64,), jnp.float32),
            pltpu.VMEM((B_PER_SUB,), jnp.float32),
            pltpu.SemaphoreType.DMA,
            pltpu.SemaphoreType.DMA,
            pltpu.SemaphoreType.DMA,
            pltpu.SemaphoreType.DMA,
        ],
    )
    p0, p1 = f(ent_t, rel_t, tail_t, h, r, t)
    out = pl.pallas_call(
        _combine_body,
        out_shape=jax.ShapeDtypeStruct((128, 128), jnp.float32),
    )(p0.reshape(128, 128), p1.reshape(128, 128))
    return out.reshape(BATCH)


def kernel(emb_ent, emb_rel, h, r, t):
    return _transe(emb_ent, emb_rel,
                   h.astype(jnp.int32), r.astype(jnp.int32),
                   t.astype(jnp.int32))
